# Initial kernel scaffold; baseline (speedup 1.0000x reference)
import jax
import jax.numpy as jnp
from jax.experimental import pallas as pl


def kernel(x, edge_index, batch, W1, b1, g1, be1, W2, b2, g2, be2, W3, b3, g3, be3, Wh1, bh1, Wh2, bh2):
    raise NotImplementedError("write your pallas kernel here")


# speedup vs baseline: 10.3419x; 10.3419x over previous
import functools

import jax
import jax.numpy as jnp
from jax import lax
from jax.experimental import pallas as pl
from jax.experimental.pallas import tpu as pltpu
from jax.experimental.pallas import tpu_sc as plsc

N = 10000
E = 320000
D = 128
G = 64
EPS = 1e-5

NC = 2
NS = 16
LANES = 16

EPC = E // NC
EPT = EPC // NS
K = 80
CHUNKS = EPT // K

ZROWS = 624
TAIL = N - NS * ZROWS
BCH = 16


def _f32(x):
    return jnp.full((LANES,), x, dtype=jnp.float32)


def _deg_body(dst_hbm, out_hbm, didx, ones_v, zb, deg_sh, sem):
    c = lax.axis_index("c")
    s = lax.axis_index("s")
    for j in range(K // LANES):
        ones_v[pl.ds(j * LANES, LANES)] = _f32(1.0)
    for j in range(ZROWS // LANES):
        zb[pl.ds(j * LANES, LANES)] = _f32(0.0)
    pltpu.sync_copy(zb, deg_sh.at[pl.ds(s * ZROWS, ZROWS)])

    @pl.when(s == 0)
    def _():
        pltpu.sync_copy(zb.at[pl.ds(0, TAIL)],
                        deg_sh.at[pl.ds(NS * ZROWS, TAIL)])

    plsc.subcore_barrier()
    base = c * EPC + s * EPT

    def chunk(i, carry):
        pltpu.sync_copy(dst_hbm.at[pl.ds(base + i * K, K)], didx)
        pltpu.sync_copy(ones_v, deg_sh.at[didx], add=True)
        return carry

    lax.fori_loop(0, CHUNKS, chunk, 0)
    plsc.subcore_barrier()
    pltpu.sync_copy(deg_sh.at[pl.ds(s * ZROWS, ZROWS)], zb)
    pltpu.sync_copy(zb, out_hbm.at[pl.ds(c * N + s * ZROWS, ZROWS)])

    @pl.when(s == 0)
    def _():
        pltpu.sync_copy(deg_sh.at[pl.ds(NS * ZROWS, TAIL)], zb.at[pl.ds(0, TAIL)])
        pltpu.sync_copy(zb.at[pl.ds(0, TAIL)],
                        out_hbm.at[pl.ds(c * N + NS * ZROWS, TAIL)])


def _make_deg():
    mesh = plsc.VectorSubcoreMesh(core_axis_name="c", subcore_axis_name="s")
    return pl.kernel(
        _deg_body,
        out_type=jax.ShapeDtypeStruct((NC * N,), jnp.float32),
        mesh=mesh,
        scratch_types=[
            pltpu.VMEM((K,), jnp.int32),
            pltpu.VMEM((K,), jnp.float32),
            pltpu.VMEM((ZROWS,), jnp.float32),
            pltpu.VMEM_SHARED((N,), jnp.float32),
            pltpu.SemaphoreType.DMA,
        ],
    )


def _scatter_body(s_hbm, src_hbm, dst_hbm, out_hbm,
                  sidx, didx, rows, zb, acc_sh, sem, F):
    c = lax.axis_index("c")
    s = lax.axis_index("s")

    for r in range(BCH):
        for j in range(F // LANES):
            zb[r, pl.ds(j * LANES, LANES)] = _f32(0.0)

    def zchunk(j, carry):
        pltpu.sync_copy(zb, acc_sh.at[pl.ds(s * ZROWS + j * BCH, BCH)])
        return carry

    lax.fori_loop(0, ZROWS // BCH, zchunk, 0)

    @pl.when(s == 0)
    def _():
        pltpu.sync_copy(zb, acc_sh.at[pl.ds(NS * ZROWS, TAIL)])

    plsc.subcore_barrier()

    base = c * EPC + s * EPT

    def chunk(i, carry):
        off = base + i * K
        pltpu.sync_copy(src_hbm.at[pl.ds(off, K)], sidx)
        pltpu.sync_copy(dst_hbm.at[pl.ds(off, K)], didx)
        pltpu.async_copy(s_hbm.at[sidx], rows, sem).wait()
        pltpu.sync_copy(rows, acc_sh.at[didx], add=True)
        return carry

    lax.fori_loop(0, CHUNKS, chunk, 0)
    plsc.subcore_barrier()

    def ochunk(j, carry):
        r0 = s * ZROWS + j * BCH
        pltpu.sync_copy(acc_sh.at[pl.ds(r0, BCH)], zb)
        pltpu.sync_copy(zb, out_hbm.at[c, pl.ds(r0, BCH)])
        return carry

    lax.fori_loop(0, ZROWS // BCH, ochunk, 0)

    @pl.when(s == 0)
    def _():
        pltpu.sync_copy(acc_sh.at[pl.ds(NS * ZROWS, TAIL)], zb)
        pltpu.sync_copy(zb, out_hbm.at[c, pl.ds(NS * ZROWS, TAIL)])


def _make_scatter(F):
    mesh = plsc.VectorSubcoreMesh(core_axis_name="c", subcore_axis_name="s")
    return pl.kernel(
        functools.partial(_scatter_body, F=F),
        out_type=jax.ShapeDtypeStruct((NC, N, F), jnp.float32),
        mesh=mesh,
        scratch_types=[
            pltpu.VMEM((K,), jnp.int32),
            pltpu.VMEM((K,), jnp.int32),
            pltpu.VMEM((K, F), jnp.float32),
            pltpu.VMEM((BCH, F), jnp.float32),
            pltpu.VMEM_SHARED((N, F), jnp.float32),
            pltpu.SemaphoreType.DMA,
        ],
    )


def _dot(a, b):
    return lax.dot_general(a, b, (((1,), (0,)), ((), ())),
                           precision=lax.Precision.HIGHEST,
                           preferred_element_type=jnp.float32)


def _tc1_body(degnt_ref, x_ref, w1_ref, dinv_ref, s1_ref):
    deg = jnp.sum(degnt_ref[...], axis=1, keepdims=True) + 1.0
    dinv = lax.rsqrt(deg)
    dinv_ref[...] = dinv
    s = _dot(x_ref[...], w1_ref[...]) * dinv
    s1_ref[...] = jnp.concatenate([s, jnp.zeros_like(s)], axis=1)


def _bn_relu(o, g, be):
    mu = jnp.mean(o, axis=0, keepdims=True)
    va = jnp.mean((o - mu) * (o - mu), axis=0, keepdims=True)
    h = (o - mu) * lax.rsqrt(va + EPS) * g + be
    return jnp.maximum(h, 0.0)


def _mid_body(acc_ref, sp_ref, dinv_ref, b_ref, g_ref, be_ref, w_ref,
              out_ref, F):
    dinv = dinv_ref[...]
    o = (acc_ref[0, :, :F] + acc_ref[1, :, :F] + sp_ref[:, :F]) * dinv \
        + b_ref[...]
    h = _bn_relu(o, g_ref[...], be_ref[...])
    s = _dot(h, w_ref[...]) * dinv
    if s.shape[1] < 128:
        s = jnp.concatenate([s, jnp.zeros_like(s)], axis=1)
    out_ref[...] = s


def _tc4_body(acc_ref, sp_ref, dinv_ref, b_ref, g_ref, be_ref, batch_ref,
              wh1_ref, bh1_ref, wh2_ref, bh2_ref, out_ref, pool_ref):
    dinv = dinv_ref[...]
    o = (acc_ref[0, :, :64] + acc_ref[1, :, :64] + sp_ref[:, :64]) * dinv \
        + b_ref[...]
    h = _bn_relu(o, g_ref[...], be_ref[...])
    bcol = batch_ref[...]
    neg = jnp.float32(-jnp.inf)

    def seg(g, carry):
        hg = jnp.where(bcol == g, h, neg)
        pool_ref[pl.ds(g, 1), :] = jnp.max(hg, axis=0, keepdims=True)
        return carry

    lax.fori_loop(0, G, seg, 0)
    pooled = pool_ref[...]
    z = jnp.maximum(_dot(pooled, wh1_ref[...]) + bh1_ref[...], 0.0)
    out_ref[...] = _dot(z, wh2_ref[...]) + bh2_ref[...]


def _tc(body, out_shape):
    return pl.pallas_call(body, out_shape=out_shape)


@jax.jit
def kernel(x, edge_index, batch, W1, b1, g1, be1, W2, b2, g2, be2,
           W3, b3, g3, be3, Wh1, bh1, Wh2, bh2):
    src = edge_index[0]
    dst = edge_index[1]
    batch_col = batch.reshape(N, 1)

    deg_parts = _make_deg()(dst).reshape(NC, N)
    deg_nt = jnp.transpose(deg_parts)

    dinv, s1 = _tc(_tc1_body, [
        jax.ShapeDtypeStruct((N, 1), jnp.float32),
        jax.ShapeDtypeStruct((N, 128), jnp.float32),
    ])(deg_nt, x, W1)

    scat = _make_scatter(128)
    acc1 = scat(s1, src, dst)
    s2 = _tc(functools.partial(_mid_body, F=64),
             jax.ShapeDtypeStruct((N, 128), jnp.float32))(
        acc1, s1, dinv, b1.reshape(1, 64), g1.reshape(1, 64),
        be1.reshape(1, 64), W2)

    acc2 = scat(s2, src, dst)
    s3 = _tc(functools.partial(_mid_body, F=128),
             jax.ShapeDtypeStruct((N, 128), jnp.float32))(
        acc2, s2, dinv, b2.reshape(1, 128), g2.reshape(1, 128),
        be2.reshape(1, 128), W3)

    acc3 = scat(s3, src, dst)
    out = pl.pallas_call(
        _tc4_body, out_shape=jax.ShapeDtypeStruct((G, 1), jnp.float32),
        scratch_shapes=[pltpu.VMEM((G, 64), jnp.float32)])(
        acc3, s3, dinv, b3.reshape(1, 64), g3.reshape(1, 64),
        be3.reshape(1, 64), batch_col, Wh1, bh1.reshape(1, 64),
        Wh2, bh2.reshape(1, 1))
    return out


# speedup vs baseline: 20.8121x; 2.0124x over previous
import functools

import jax
import jax.numpy as jnp
from jax import lax
from jax.experimental import pallas as pl
from jax.experimental.pallas import tpu as pltpu
from jax.experimental.pallas import tpu_sc as plsc

N = 10000
E = 320000
D = 128
G = 64
EPS = 1e-5

NC = 2
NS = 16
LANES = 16

EPC = E // NC
EPT = EPC // NS
K = 80
CHUNKS = EPT // K

ZROWS = 624
TAIL = N - NS * ZROWS
BCH = 104
NSLOT = 4


def _f32(x):
    return jnp.full((LANES,), x, dtype=jnp.float32)


def _deg_body(dst_hbm, out_hbm, didx, ones_v, zb, deg_sh, sem):
    c = lax.axis_index("c")
    s = lax.axis_index("s")
    for j in range(K // LANES):
        ones_v[pl.ds(j * LANES, LANES)] = _f32(1.0)
    for j in range(ZROWS // LANES):
        zb[pl.ds(j * LANES, LANES)] = _f32(0.0)
    pltpu.sync_copy(zb, deg_sh.at[pl.ds(s * ZROWS, ZROWS)])

    @pl.when(s == 0)
    def _():
        pltpu.sync_copy(zb.at[pl.ds(0, TAIL)],
                        deg_sh.at[pl.ds(NS * ZROWS, TAIL)])

    plsc.subcore_barrier()
    base = c * EPC + s * EPT

    def chunk(i, carry):
        pltpu.sync_copy(dst_hbm.at[pl.ds(base + i * K, K)], didx)
        pltpu.sync_copy(ones_v, deg_sh.at[didx], add=True)
        return carry

    lax.fori_loop(0, CHUNKS, chunk, 0)
    plsc.subcore_barrier()
    pltpu.sync_copy(deg_sh.at[pl.ds(s * ZROWS, ZROWS)], zb)
    pltpu.sync_copy(zb, out_hbm.at[pl.ds(c * N + s * ZROWS, ZROWS)])

    @pl.when(s == 0)
    def _():
        pltpu.sync_copy(deg_sh.at[pl.ds(NS * ZROWS, TAIL)], zb.at[pl.ds(0, TAIL)])
        pltpu.sync_copy(zb.at[pl.ds(0, TAIL)],
                        out_hbm.at[pl.ds(c * N + NS * ZROWS, TAIL)])


def _make_deg():
    mesh = plsc.VectorSubcoreMesh(core_axis_name="c", subcore_axis_name="s")
    return pl.kernel(
        _deg_body,
        out_type=jax.ShapeDtypeStruct((NC * N,), jnp.float32),
        mesh=mesh,
        scratch_types=[
            pltpu.VMEM((K,), jnp.int32),
            pltpu.VMEM((K,), jnp.float32),
            pltpu.VMEM((ZROWS,), jnp.float32),
            pltpu.VMEM_SHARED((N,), jnp.float32),
            pltpu.SemaphoreType.DMA,
        ],
    )


def _scatter_body(s_hbm, src_hbm, dst_hbm, out_hbm,
                  sidx, didx, rows, zb, acc_sh, gsem, isem0, isem1, osem, F):
    c = lax.axis_index("c")
    s = lax.axis_index("s")
    base = c * EPC + s * EPT

    def fire_idx(j):
        sl = lax.rem(j, NSLOT)
        sem = isem0 if isinstance(j, int) and j % 2 == 0 else None
        off = base + j * K
        if sem is None:
            @pl.when(lax.rem(j, 2) == 0)
            def _():
                pltpu.async_copy(src_hbm.at[pl.ds(off, K)], sidx.at[sl], isem0)
                pltpu.async_copy(dst_hbm.at[pl.ds(off, K)], didx.at[sl], isem0)

            @pl.when(lax.rem(j, 2) == 1)
            def _():
                pltpu.async_copy(src_hbm.at[pl.ds(off, K)], sidx.at[sl], isem1)
                pltpu.async_copy(dst_hbm.at[pl.ds(off, K)], didx.at[sl], isem1)
        else:
            pltpu.async_copy(src_hbm.at[pl.ds(off, K)], sidx.at[sl], isem0)
            pltpu.async_copy(dst_hbm.at[pl.ds(off, K)], didx.at[sl], isem0)

    def wait_idx(j):
        @pl.when(lax.rem(j, 2) == 0)
        def _():
            pltpu.make_async_copy(src_hbm.at[pl.ds(0, K)], sidx.at[0], isem0).wait()
            pltpu.make_async_copy(dst_hbm.at[pl.ds(0, K)], didx.at[0], isem0).wait()

        @pl.when(lax.rem(j, 2) == 1)
        def _():
            pltpu.make_async_copy(src_hbm.at[pl.ds(0, K)], sidx.at[0], isem1).wait()
            pltpu.make_async_copy(dst_hbm.at[pl.ds(0, K)], didx.at[0], isem1).wait()

    fire_idx(0)

    for r in range(BCH):
        for j in range(F // LANES):
            zb[0, r, pl.ds(j * LANES, LANES)] = _f32(0.0)

    wait_idx(0)
    pltpu.async_copy(s_hbm.at[sidx.at[0]], rows.at[0], gsem)

    def fire1():
        off = base + K
        pltpu.async_copy(src_hbm.at[pl.ds(off, K)], sidx.at[1], isem1)
        pltpu.async_copy(dst_hbm.at[pl.ds(off, K)], didx.at[1], isem1)

    fire1()

    for q in range(ZROWS // BCH):
        pltpu.async_copy(zb.at[0], acc_sh.at[pl.ds(s * ZROWS + q * BCH, BCH)],
                         osem)

    @pl.when(s == 0)
    def _():
        pltpu.async_copy(zb.at[0, pl.ds(0, TAIL)],
                         acc_sh.at[pl.ds(NS * ZROWS, TAIL)], osem)

    for q in range(ZROWS // BCH):
        pltpu.make_async_copy(zb.at[0], acc_sh.at[pl.ds(0, BCH)], osem).wait()

    @pl.when(s == 0)
    def _():
        pltpu.make_async_copy(zb.at[0, pl.ds(0, TAIL)],
                              acc_sh.at[pl.ds(0, TAIL)], osem).wait()

    plsc.subcore_barrier()

    def step(i, carry):
        sl = lax.rem(i, NSLOT)
        sl1 = lax.rem(i + 1, NSLOT)
        rb = lax.rem(i, 2)
        rb1 = lax.rem(i + 1, 2)

        @pl.when(i + 1 < CHUNKS)
        def _():
            wait_idx(i + 1)
            pltpu.async_copy(s_hbm.at[sidx.at[sl1]], rows.at[rb1], gsem)

        @pl.when(i + 2 < CHUNKS)
        def _():
            fire_idx(i + 2)

        pltpu.make_async_copy(s_hbm.at[sidx.at[sl]], rows.at[rb], gsem).wait()
        pltpu.sync_copy(rows.at[rb], acc_sh.at[didx.at[sl]], add=True)
        return carry

    lax.fori_loop(0, CHUNKS, step, 0)
    plsc.subcore_barrier()

    def ochunk(q, carry):
        b = lax.rem(q, 2)

        @pl.when(q >= 2)
        def _():
            pltpu.make_async_copy(zb.at[0], out_hbm.at[c, pl.ds(0, BCH)],
                                  osem).wait()

        r0 = s * ZROWS + q * BCH
        pltpu.sync_copy(acc_sh.at[pl.ds(r0, BCH)], zb.at[b])
        pltpu.async_copy(zb.at[b], out_hbm.at[c, pl.ds(r0, BCH)], osem)
        return carry

    lax.fori_loop(0, ZROWS // BCH, ochunk, 0)
    pltpu.make_async_copy(zb.at[0], out_hbm.at[c, pl.ds(0, BCH)], osem).wait()
    pltpu.make_async_copy(zb.at[0], out_hbm.at[c, pl.ds(0, BCH)], osem).wait()

    @pl.when(s == 0)
    def _():
        pltpu.sync_copy(acc_sh.at[pl.ds(NS * ZROWS, TAIL)],
                        zb.at[0, pl.ds(0, TAIL)])
        pltpu.sync_copy(zb.at[0, pl.ds(0, TAIL)],
                        out_hbm.at[c, pl.ds(NS * ZROWS, TAIL)])


def _make_scatter(F):
    mesh = plsc.VectorSubcoreMesh(core_axis_name="c", subcore_axis_name="s")
    return pl.kernel(
        functools.partial(_scatter_body, F=F),
        out_type=jax.ShapeDtypeStruct((NC, N, F), jnp.float32),
        mesh=mesh,
        scratch_types=[
            pltpu.VMEM((NSLOT, K), jnp.int32),
            pltpu.VMEM((NSLOT, K), jnp.int32),
            pltpu.VMEM((2, K, F), jnp.float32),
            pltpu.VMEM((2, BCH, F), jnp.float32),
            pltpu.VMEM_SHARED((N, F), jnp.float32),
            pltpu.SemaphoreType.DMA,
            pltpu.SemaphoreType.DMA,
            pltpu.SemaphoreType.DMA,
            pltpu.SemaphoreType.DMA,
        ],
    )


def _dot(a, b):
    return lax.dot_general(a, b, (((1,), (0,)), ((), ())),
                           precision=lax.Precision.HIGHEST,
                           preferred_element_type=jnp.float32)


def _tc1_body(degnt_ref, x_ref, w1_ref, dinv_ref, s1_ref):
    deg = jnp.sum(degnt_ref[...], axis=1, keepdims=True) + 1.0
    dinv = lax.rsqrt(deg)
    dinv_ref[...] = dinv
    s = _dot(x_ref[...], w1_ref[...]) * dinv
    s1_ref[...] = jnp.concatenate([s, jnp.zeros_like(s)], axis=1)


def _bn_relu(o, g, be):
    mu = jnp.mean(o, axis=0, keepdims=True)
    va = jnp.mean((o - mu) * (o - mu), axis=0, keepdims=True)
    h = (o - mu) * lax.rsqrt(va + EPS) * g + be
    return jnp.maximum(h, 0.0)


def _mid_body(acc_ref, sp_ref, dinv_ref, b_ref, g_ref, be_ref, w_ref,
              out_ref, F):
    dinv = dinv_ref[...]
    o = (acc_ref[0, :, :F] + acc_ref[1, :, :F] + sp_ref[:, :F]) * dinv \
        + b_ref[...]
    h = _bn_relu(o, g_ref[...], be_ref[...])
    s = _dot(h, w_ref[...]) * dinv
    if s.shape[1] < 128:
        s = jnp.concatenate([s, jnp.zeros_like(s)], axis=1)
    out_ref[...] = s


def _tc4_body(acc_ref, sp_ref, dinv_ref, b_ref, g_ref, be_ref, batch_ref,
              wh1_ref, bh1_ref, wh2_ref, bh2_ref, out_ref, pool_ref):
    dinv = dinv_ref[...]
    o = (acc_ref[0, :, :64] + acc_ref[1, :, :64] + sp_ref[:, :64]) * dinv \
        + b_ref[...]
    h = _bn_relu(o, g_ref[...], be_ref[...])
    bcol = batch_ref[...]
    neg = jnp.float32(-jnp.inf)

    def seg(g, carry):
        hg = jnp.where(bcol == g, h, neg)
        pool_ref[pl.ds(g, 1), :] = jnp.max(hg, axis=0, keepdims=True)
        return carry

    lax.fori_loop(0, G, seg, 0)
    pooled = pool_ref[...]
    z = jnp.maximum(_dot(pooled, wh1_ref[...]) + bh1_ref[...], 0.0)
    out_ref[...] = _dot(z, wh2_ref[...]) + bh2_ref[...]


def _tc(body, out_shape):
    return pl.pallas_call(body, out_shape=out_shape)


@jax.jit
def kernel(x, edge_index, batch, W1, b1, g1, be1, W2, b2, g2, be2,
           W3, b3, g3, be3, Wh1, bh1, Wh2, bh2):
    src = edge_index[0]
    dst = edge_index[1]
    batch_col = batch.reshape(N, 1)

    deg_parts = _make_deg()(dst).reshape(NC, N)
    deg_nt = jnp.transpose(deg_parts)

    dinv, s1 = _tc(_tc1_body, [
        jax.ShapeDtypeStruct((N, 1), jnp.float32),
        jax.ShapeDtypeStruct((N, 128), jnp.float32),
    ])(deg_nt, x, W1)

    scat = _make_scatter(128)
    acc1 = scat(s1, src, dst)
    s2 = _tc(functools.partial(_mid_body, F=64),
             jax.ShapeDtypeStruct((N, 128), jnp.float32))(
        acc1, s1, dinv, b1.reshape(1, 64), g1.reshape(1, 64),
        be1.reshape(1, 64), W2)

    acc2 = scat(s2, src, dst)
    s3 = _tc(functools.partial(_mid_body, F=128),
             jax.ShapeDtypeStruct((N, 128), jnp.float32))(
        acc2, s2, dinv, b2.reshape(1, 128), g2.reshape(1, 128),
        be2.reshape(1, 128), W3)

    acc3 = scat(s3, src, dst)
    out = pl.pallas_call(
        _tc4_body, out_shape=jax.ShapeDtypeStruct((G, 1), jnp.float32),
        scratch_shapes=[pltpu.VMEM((G, 64), jnp.float32)])(
        acc3, s3, dinv, b3.reshape(1, 64), g3.reshape(1, 64),
        be3.reshape(1, 64), batch_col, Wh1, bh1.reshape(1, 64),
        Wh2, bh2.reshape(1, 1))
    return out
